# baseline (speedup 1.0000x reference)
import functools

import jax
import jax.numpy as jnp
from jax import lax
from jax.experimental import pallas as pl
from jax.experimental.pallas import tpu as pltpu
from jax.experimental.pallas import tpu_sc as plsc

N_USER = 2000
N_ITEM = 8000
N_NODES = N_USER + N_ITEM
DIM = 128
N_EDGES = 320000

NC = 2
NS = 16
CHUNK = 128
CHUNKS_PER_WORKER = 80
HALVES = 2
HALF_CHUNKS = CHUNKS_PER_WORKER // HALVES
NPAIRS = HALF_CHUNKS // 2
E_PAD = NC * NS * CHUNKS_PER_WORKER * CHUNK
ACC_ROWS = 10240
ROWS_PER_SUB = ACC_ROWS // NS
ZROWS = 16

_sc_mesh = plsc.VectorSubcoreMesh(core_axis_name="c", subcore_axis_name="s")


@functools.partial(
    pl.kernel,
    out_type=jax.ShapeDtypeStruct((NC, ACC_ROWS, DIM), jnp.float32),
    mesh=_sc_mesh,
    scratch_types=[
        pltpu.VMEM((HALF_CHUNKS, CHUNK), jnp.int32),
        pltpu.VMEM((HALF_CHUNKS, CHUNK), jnp.int32),
        pltpu.VMEM((CHUNK, DIM), jnp.float32),
        pltpu.VMEM((CHUNK, DIM), jnp.float32),
        pltpu.VMEM((ZROWS, DIM), jnp.float32),
        pltpu.VMEM_SHARED((ACC_ROWS, DIM), jnp.float32),
        pltpu.SemaphoreType.DMA,
        pltpu.SemaphoreType.DMA,
    ],
)
def _sc_aggregate(xn_hbm, src_hbm, dst_hbm, out_hbm,
                  src_v, dst_v, rows_a, rows_b, zero_v, acc_sh,
                  sem_a, sem_b):
    cid = lax.axis_index("c")
    sid = lax.axis_index("s")

    @pl.loop(0, ZROWS)
    def _(r):
        @pl.loop(0, DIM, step=16)
        def _(q):
            zero_v[pl.ds(r, 1), pl.ds(q, 16)] = jnp.zeros((1, 16), jnp.float32)

    @pl.loop(0, ROWS_PER_SUB, step=ZROWS)
    def _(r):
        pltpu.sync_copy(zero_v, acc_sh.at[pl.ds(sid * ROWS_PER_SUB + r, ZROWS)])

    plsc.subcore_barrier()

    wid = cid * NS + sid

    @pl.loop(0, HALVES)
    def _(h):
        crow = wid * CHUNKS_PER_WORKER + h * HALF_CHUNKS
        pltpu.sync_copy(src_hbm.at[pl.ds(crow, HALF_CHUNKS)], src_v)
        pltpu.sync_copy(dst_hbm.at[pl.ds(crow, HALF_CHUNKS)], dst_v)

        @pl.loop(0, HALF_CHUNKS)
        def _(i):
            pltpu.sync_copy(xn_hbm.at[src_v.at[i]], rows_a)
            pltpu.sync_copy(rows_a, acc_sh.at[dst_v.at[i]], add=True)

    plsc.subcore_barrier()

    pltpu.sync_copy(acc_sh.at[pl.ds(sid * ROWS_PER_SUB, ROWS_PER_SUB)],
                    out_hbm.at[cid, pl.ds(sid * ROWS_PER_SUB, ROWS_PER_SUB)])


def _normalize_body(x_ref, o_ref):
    x = x_ref[...]
    s = jnp.sum(x * x, axis=1, keepdims=True)
    norm = jnp.sqrt(s)
    o_ref[...] = x / jnp.maximum(norm, 1e-12)


def _combine_body(p0_ref, p1_ref, xn_ref, w_ref, o_ref):
    s = p0_ref[0] + p1_ref[0]
    o_ref[...] = (
        jnp.dot(s, w_ref[...], preferred_element_type=jnp.float32)
        + xn_ref[...]
    )


_ROWB = 1000


def kernel(edge_index, features, preference, W):
    x = jnp.concatenate([preference, features], axis=0)
    src = edge_index[0]
    dst = edge_index[1]
    pad = E_PAD - N_EDGES
    n_crows = NC * NS * CHUNKS_PER_WORKER
    src_p = jnp.concatenate(
        [src, jnp.zeros((pad,), jnp.int32)]).reshape(n_crows, CHUNK)
    dst_p = jnp.concatenate(
        [dst, jnp.full((pad,), N_NODES, jnp.int32)]).reshape(n_crows, CHUNK)

    xn = pl.pallas_call(
        _normalize_body,
        out_shape=jax.ShapeDtypeStruct((N_NODES, DIM), jnp.float32),
        grid=(N_NODES // _ROWB,),
        in_specs=[pl.BlockSpec((_ROWB, DIM), lambda i: (i, 0))],
        out_specs=pl.BlockSpec((_ROWB, DIM), lambda i: (i, 0)),
    )(x)

    parts = _sc_aggregate(xn, src_p, dst_p)

    x_hat = pl.pallas_call(
        _combine_body,
        out_shape=jax.ShapeDtypeStruct((N_NODES, DIM), jnp.float32),
        grid=(N_NODES // _ROWB,),
        in_specs=[
            pl.BlockSpec((1, _ROWB, DIM), lambda i: (0, i, 0)),
            pl.BlockSpec((1, _ROWB, DIM), lambda i: (1, i, 0)),
            pl.BlockSpec((_ROWB, DIM), lambda i: (i, 0)),
            pl.BlockSpec((DIM, DIM), lambda i: (0, 0)),
        ],
        out_specs=pl.BlockSpec((_ROWB, DIM), lambda i: (i, 0)),
    )(parts, parts, xn, W)

    return (x_hat, preference)


# speedup vs baseline: 1.1569x; 1.1569x over previous
import functools

import jax
import jax.numpy as jnp
from jax import lax
from jax.experimental import pallas as pl
from jax.experimental.pallas import tpu as pltpu
from jax.experimental.pallas import tpu_sc as plsc

N_USER = 2000
N_ITEM = 8000
N_NODES = N_USER + N_ITEM
DIM = 128
N_EDGES = 320000

NC = 2
NS = 16
CHUNK = 128
CHUNKS_C0 = 116
CHUNKS_C1 = 44
TOTAL_CHUNKS = NS * (CHUNKS_C0 + CHUNKS_C1)
E_PAD = TOTAL_CHUNKS * CHUNK
ACC_ROWS = 10240
ROWS_PER_SUB = ACC_ROWS // NS
ZROWS = 16

_sc_mesh = plsc.VectorSubcoreMesh(core_axis_name="c", subcore_axis_name="s")


@functools.partial(
    pl.kernel,
    out_type=jax.ShapeDtypeStruct((NC, ACC_ROWS, DIM), jnp.float32),
    mesh=_sc_mesh,
    scratch_types=[
        pltpu.VMEM((CHUNK,), jnp.int32),
        pltpu.VMEM((CHUNK,), jnp.int32),
        pltpu.VMEM((CHUNK,), jnp.int32),
        pltpu.VMEM((CHUNK,), jnp.int32),
        pltpu.VMEM((CHUNK, DIM), jnp.float32),
        pltpu.VMEM((CHUNK, DIM), jnp.float32),
        pltpu.VMEM((ZROWS, DIM), jnp.float32),
        pltpu.VMEM_SHARED((ACC_ROWS, DIM), jnp.float32),
        pltpu.SemaphoreType.DMA,
        pltpu.SemaphoreType.DMA,
    ],
)
def _sc_aggregate(xn_hbm, src_hbm, dst_hbm, out_hbm,
                  src_a, dst_a, src_b, dst_b, rows_a, rows_b, zero_v, acc_sh,
                  sem_a, sem_b):
    cid = lax.axis_index("c")
    sid = lax.axis_index("s")

    @pl.loop(0, ZROWS)
    def _(r):
        @pl.loop(0, DIM, step=16)
        def _(q):
            zero_v[pl.ds(r, 1), pl.ds(q, 16)] = jnp.zeros((1, 16), jnp.float32)

    @pl.loop(0, ROWS_PER_SUB, step=ZROWS)
    def _(r):
        pltpu.sync_copy(zero_v, acc_sh.at[pl.ds(sid * ROWS_PER_SUB + r, ZROWS)])

    plsc.subcore_barrier()

    n_chunks = jnp.where(cid == 0, CHUNKS_C0, CHUNKS_C1)
    base = jnp.where(cid == 0,
                     sid * CHUNKS_C0,
                     NS * CHUNKS_C0 + sid * CHUNKS_C1) * CHUNK

    def _load_idx(i, s_v, d_v):
        off = base + i * CHUNK
        pltpu.sync_copy(src_hbm.at[pl.ds(off, CHUNK)], s_v)
        pltpu.sync_copy(dst_hbm.at[pl.ds(off, CHUNK)], d_v)

    _load_idx(0, src_a, dst_a)
    pltpu.async_copy(xn_hbm.at[src_a], rows_a, sem_a)

    def _pair(k, carry):
        ia = 2 * k
        _load_idx(ia + 1, src_b, dst_b)
        pltpu.async_copy(xn_hbm.at[src_b], rows_b, sem_b)
        pltpu.make_async_copy(xn_hbm.at[src_a], rows_a, sem_a).wait()
        pltpu.sync_copy(rows_a, acc_sh.at[dst_a], add=True)

        @pl.when(ia + 2 < n_chunks)
        def _():
            _load_idx(ia + 2, src_a, dst_a)
            pltpu.async_copy(xn_hbm.at[src_a], rows_a, sem_a)

        pltpu.make_async_copy(xn_hbm.at[src_b], rows_b, sem_b).wait()
        pltpu.sync_copy(rows_b, acc_sh.at[dst_b], add=True)
        return carry

    lax.fori_loop(0, n_chunks // 2, _pair, 0)

    plsc.subcore_barrier()

    pltpu.sync_copy(acc_sh.at[pl.ds(sid * ROWS_PER_SUB, ROWS_PER_SUB)],
                    out_hbm.at[cid, pl.ds(sid * ROWS_PER_SUB, ROWS_PER_SUB)])


def _normalize_body(x_ref, o_ref):
    x = x_ref[...]
    s = jnp.sum(x * x, axis=1, keepdims=True)
    norm = jnp.sqrt(s)
    o_ref[...] = x / jnp.maximum(norm, 1e-12)


def _combine_body(p0_ref, p1_ref, xn_ref, w_ref, o_ref):
    s = p0_ref[0] + p1_ref[0]
    o_ref[...] = (
        jnp.dot(s, w_ref[...], preferred_element_type=jnp.float32)
        + xn_ref[...]
    )


_ROWB = 1000


def kernel(edge_index, features, preference, W):
    x = jnp.concatenate([preference, features], axis=0)
    src = edge_index[0]
    dst = edge_index[1]
    pad = E_PAD - N_EDGES
    src_p = jnp.concatenate([src, jnp.zeros((pad,), jnp.int32)])
    dst_p = jnp.concatenate([dst, jnp.full((pad,), N_NODES, jnp.int32)])

    xn = pl.pallas_call(
        _normalize_body,
        out_shape=jax.ShapeDtypeStruct((N_NODES, DIM), jnp.float32),
        grid=(N_NODES // _ROWB,),
        in_specs=[pl.BlockSpec((_ROWB, DIM), lambda i: (i, 0))],
        out_specs=pl.BlockSpec((_ROWB, DIM), lambda i: (i, 0)),
    )(x)

    parts = _sc_aggregate(xn, src_p, dst_p)

    x_hat = pl.pallas_call(
        _combine_body,
        out_shape=jax.ShapeDtypeStruct((N_NODES, DIM), jnp.float32),
        grid=(N_NODES // _ROWB,),
        in_specs=[
            pl.BlockSpec((1, _ROWB, DIM), lambda i: (0, i, 0)),
            pl.BlockSpec((1, _ROWB, DIM), lambda i: (1, i, 0)),
            pl.BlockSpec((_ROWB, DIM), lambda i: (i, 0)),
            pl.BlockSpec((DIM, DIM), lambda i: (0, 0)),
        ],
        out_specs=pl.BlockSpec((_ROWB, DIM), lambda i: (i, 0)),
    )(parts, parts, xn, W)

    return (x_hat, preference)
